# Initial kernel scaffold; baseline (speedup 1.0000x reference)
import jax
import jax.numpy as jnp
from jax.experimental import pallas as pl


def kernel(x, edge_index, W_self_0, W_neigh_0, b_0, gamma_0, beta_0, W_self_1, W_neigh_1, b_1, gamma_1, beta_1):
    raise NotImplementedError("write your pallas kernel here")


# speedup vs baseline: 3.8144x; 3.8144x over previous
import dataclasses

import jax
import jax.numpy as jnp
from jax import lax
from jax.experimental import pallas as pl
from jax.experimental.pallas import tpu as pltpu
from jax.experimental.pallas import tpu_sc as plsc

N = 10000
D = 128
NC = 2
NS = 16
NW = NC * NS
NPAD = 10240
ROWS_PER_SUB = NPAD // NS
CHUNK = 64
STAGE = 64
LANES = 16


def _sc_agg(x, src_p, dst_p, zrows, zdeg, n_chunks, with_deg):
    mesh = plsc.VectorSubcoreMesh(core_axis_name="c", subcore_axis_name="s")
    out_type = [jax.ShapeDtypeStruct((NC * NPAD, D), jnp.float32)]
    scratch = [
        pltpu.VMEM_SHARED((NPAD, D), jnp.float32),
        pltpu.VMEM((CHUNK,), jnp.int32),
        pltpu.VMEM((CHUNK,), jnp.int32),
        pltpu.VMEM((CHUNK, D), jnp.float32),
        pltpu.VMEM((STAGE, D), jnp.float32),
    ]
    if with_deg:
        out_type.append(jax.ShapeDtypeStruct((NW, NPAD), jnp.float32))
        scratch.append(pltpu.VMEM((NPAD,), jnp.float32))
    e_per_w = n_chunks * CHUNK

    def body(x_hbm, src_hbm, dst_hbm, zr_hbm, zd_hbm, *rest):
        if with_deg:
            agg_hbm, deg_hbm, acc, src_v, dst_v, rows_v, stage_v, dacc_v = rest
        else:
            agg_hbm, acc, src_v, dst_v, rows_v, stage_v = rest
        c = lax.axis_index("c")
        s = lax.axis_index("s")
        wid = c * NS + s
        r0 = s * ROWS_PER_SUB
        ebase = wid * e_per_w

        pltpu.sync_copy(zr_hbm, stage_v)
        if with_deg:
            pltpu.sync_copy(zd_hbm, dacc_v)

        @pl.loop(0, ROWS_PER_SUB // STAGE)
        def _(j):
            pltpu.sync_copy(stage_v, acc.at[pl.ds(r0 + j * STAGE, STAGE)])

        plsc.subcore_barrier()

        ones16 = jnp.ones((LANES,), jnp.float32)

        @pl.loop(0, n_chunks)
        def _(k):
            off = pl.multiple_of(ebase + k * CHUNK, 8)
            pltpu.sync_copy(src_hbm.at[pl.ds(off, CHUNK)], src_v)
            pltpu.sync_copy(dst_hbm.at[pl.ds(off, CHUNK)], dst_v)
            pltpu.sync_copy(x_hbm.at[src_v], rows_v)
            pltpu.sync_copy(rows_v, acc.at[dst_v], add=True)
            if with_deg:
                for g in range(CHUNK // LANES):
                    idx16 = dst_v[pl.ds(g * LANES, LANES)]
                    plsc.addupdate_scatter(dacc_v, [idx16], ones16)

        plsc.subcore_barrier()

        @pl.loop(0, ROWS_PER_SUB // STAGE)
        def _(j):
            ro = r0 + j * STAGE
            pltpu.sync_copy(acc.at[pl.ds(ro, STAGE)], stage_v)
            pltpu.sync_copy(stage_v, agg_hbm.at[pl.ds(c * NPAD + ro, STAGE)])

        if with_deg:
            pltpu.sync_copy(dacc_v, deg_hbm.at[wid])

    cp = pltpu.CompilerParams()
    if "needs_layout_passes" in pltpu.CompilerParams.__dataclass_fields__:
        cp = dataclasses.replace(cp, needs_layout_passes=False)
    k = pl.kernel(body, out_type=out_type, mesh=mesh, scratch_types=scratch,
                  compiler_params=cp)
    return k(x, src_p, dst_p, zrows, zdeg)


TC_BLK = 1000
DEGW = 32


def _tc_layer(h, agg_a, agg_b, deg_t, Ws, Wn, b, g, be, last):
    def pre_body(h_ref, aa_ref, ab_ref, dt_ref, ws_ref, wn_ref,
                 b_ref, o_ref):
        agg = aa_ref[...] + ab_ref[...]
        deg = jnp.sum(dt_ref[...], axis=1, keepdims=True)
        neigh = agg / jnp.maximum(deg, 1.0)
        o_ref[...] = (
            jnp.dot(h_ref[...], ws_ref[...],
                    preferred_element_type=jnp.float32,
                    precision=lax.Precision.HIGHEST)
            + jnp.dot(neigh, wn_ref[...],
                      preferred_element_type=jnp.float32,
                      precision=lax.Precision.HIGHEST)
            + b_ref[...]
        )

    row_blk = lambda: pl.BlockSpec((TC_BLK, D), lambda i: (i, 0))
    deg_blk = lambda: pl.BlockSpec((TC_BLK, DEGW), lambda i: (i, 0))
    full = lambda shape: pl.BlockSpec(shape, lambda i: (0, 0))
    pre = pl.pallas_call(
        pre_body,
        grid=(N // TC_BLK,),
        in_specs=[row_blk(), row_blk(), row_blk(), deg_blk(),
                  full((D, D)), full((D, D)), full((1, D))],
        out_specs=row_blk(),
        out_shape=jax.ShapeDtypeStruct((N, D), jnp.float32),
    )(h, agg_a, agg_b, deg_t, Ws, Wn, b)

    def bn_body(p_ref, g_ref, be_ref, o_ref):
        out = p_ref[...]
        mu = jnp.mean(out, axis=0, keepdims=True)
        var = jnp.mean((out - mu) ** 2, axis=0, keepdims=True)
        xn = (out - mu) * lax.rsqrt(var + 1e-5)
        out = g_ref[...] * xn + be_ref[...]
        if last:
            o_ref[...] = jax.nn.sigmoid(out)
        else:
            o_ref[...] = jnp.maximum(out, 0.0)

    return pl.pallas_call(
        bn_body, out_shape=jax.ShapeDtypeStruct((N, D), jnp.float32)
    )(pre, g, be)


def kernel(x, edge_index, W_self_0, W_neigh_0, b_0, gamma_0, beta_0,
           W_self_1, W_neigh_1, b_1, gamma_1, beta_1):
    E = edge_index.shape[1]
    src = edge_index[0].astype(jnp.int32)
    dst = edge_index[1].astype(jnp.int32)
    epad = -E % (NW * CHUNK)
    if epad:
        src = jnp.concatenate([src, jnp.zeros((epad,), jnp.int32)])
        dst = jnp.concatenate(
            [dst, N + (jnp.arange(epad, dtype=jnp.int32) % (NPAD - N))])
    n_chunks = (E + epad) // (NW * CHUNK)

    zrows = jnp.zeros((STAGE, D), jnp.float32)
    zdeg = jnp.zeros((NPAD,), jnp.float32)

    aggp1, degp = _sc_agg(x, src, dst, zrows, zdeg, n_chunks, True)
    agg1_a = lax.slice(aggp1, (0, 0), (N, D))
    agg1_b = lax.slice(aggp1, (NPAD, 0), (NPAD + N, D))
    deg_t = lax.slice(jnp.transpose(degp), (0, 0), (N, NW))

    h1 = _tc_layer(x, agg1_a, agg1_b, deg_t,
                   W_self_0, W_neigh_0, b_0.reshape(1, D),
                   gamma_0.reshape(1, D), beta_0.reshape(1, D), False)

    (aggp2,) = _sc_agg(h1, src, dst, zrows, zdeg, n_chunks, False)
    agg2_a = lax.slice(aggp2, (0, 0), (N, D))
    agg2_b = lax.slice(aggp2, (NPAD, 0), (NPAD + N, D))

    return _tc_layer(h1, agg2_a, agg2_b, deg_t,
                     W_self_1, W_neigh_1, b_1.reshape(1, D),
                     gamma_1.reshape(1, D), beta_1.reshape(1, D), True)
